# baseline (speedup 1.0000x reference)
import functools

import jax
import jax.numpy as jnp
from jax import lax
from jax.experimental import pallas as pl
from jax.experimental.pallas import tpu as pltpu
from jax.experimental.pallas import tpu_sc as plsc

_CHUNK = 128
_LANES = 16


@functools.cache
def _build(B, V, D):
    fold = 128 // D
    shift = fold.bit_length() - 1
    info = plsc.get_sparse_core_info()
    n_workers = info.num_cores * info.num_subcores
    b_per_w = B // n_workers
    n_chunks = b_per_w // _CHUNK
    n_groups = b_per_w // _LANES
    mesh = plsc.VectorSubcoreMesh(core_axis_name="c", subcore_axis_name="s")

    @functools.partial(
        pl.kernel,
        mesh=mesh,
        out_type=jax.ShapeDtypeStruct((B, D), jnp.float32),
        scratch_types=[
            pltpu.VMEM((b_per_w,), jnp.int32),
            pltpu.VMEM((b_per_w,), jnp.int32),
            pltpu.VMEM((_CHUNK, 128), jnp.float32),
            pltpu.VMEM((_CHUNK, 128), jnp.float32),
            pltpu.VMEM((b_per_w, D), jnp.float32),
            pltpu.SemaphoreType.DMA,
            pltpu.SemaphoreType.DMA,
        ],
        compiler_params=pltpu.CompilerParams(needs_layout_passes=False),
    )
    def gather_kernel(idx_hbm, table_hbm, out_hbm, idx_v, row_v, buf_a,
                      buf_b, out_v, sem_a, sem_b):
        wid = lax.axis_index("s") * info.num_cores + lax.axis_index("c")
        base = wid * b_per_w
        pltpu.sync_copy(idx_hbm.at[pl.ds(base, b_per_w)], idx_v)

        def row_body(g, carry):
            s = g * _LANES
            row_v[pl.ds(s, _LANES)] = idx_v[pl.ds(s, _LANES)] >> shift
            return carry

        lax.fori_loop(0, n_groups, row_body, 0)

        bufs = [buf_a, buf_b]
        sems = [sem_a, sem_b]

        def fire(j):
            return pltpu.async_copy(
                table_hbm.at[row_v.at[pl.ds(j * _CHUNK, _CHUNK)]],
                bufs[j % 2], sems[j % 2])

        lane_iota = lax.iota(jnp.int32, _LANES)

        def extract(buf, chunk_base):
            def body(g, carry):
                s = chunk_base + g * _LANES
                r16 = lane_iota + (g * _LANES)
                off = (idx_v[pl.ds(s, _LANES)] & (fold - 1)) * D
                o16 = lane_iota + s
                for j in range(D):
                    vals = plsc.load_gather(buf, [r16, off + j])
                    plsc.store_scatter(
                        out_v, [o16, jnp.full((_LANES,), j, jnp.int32)], vals)
                return carry

            lax.fori_loop(0, _CHUNK // _LANES, body, 0)

        copies = [fire(0), fire(1)]
        for j in range(n_chunks):
            copies[j].wait()
            extract(bufs[j % 2], j * _CHUNK)
            if j + 2 < n_chunks:
                copies.append(fire(j + 2))
        pltpu.sync_copy(out_v, out_hbm.at[pl.ds(base, b_per_w)])

    return gather_kernel


def kernel(class_labels, embedding):
    B, = class_labels.shape
    V, D = embedding.shape
    folded = embedding.reshape(V * D // 128, 128)
    return _build(B, V, D)(class_labels.astype(jnp.int32), folded)


# speedup vs baseline: 3.8680x; 3.8680x over previous
import functools

import jax
import jax.numpy as jnp
from jax import lax
from jax.experimental import pallas as pl
from jax.experimental.pallas import tpu as pltpu
from jax.experimental.pallas import tpu_sc as plsc

_K = 8
_LANES = 16


@functools.cache
def _build(B, V, D):
    info = plsc.get_sparse_core_info()
    n_workers = info.num_cores * info.num_subcores
    b_per_w = B // n_workers
    n_batches = b_per_w // _K
    mesh = plsc.VectorSubcoreMesh(core_axis_name="c", subcore_axis_name="s")

    @functools.partial(
        pl.kernel,
        mesh=mesh,
        out_type=jax.ShapeDtypeStruct((D, B), jnp.float32),
        scratch_types=[
            pltpu.VMEM((b_per_w,), jnp.int32),
            pltpu.VMEM((D, _K * 128), jnp.float32),
            pltpu.VMEM((D, _K * 128), jnp.float32),
            pltpu.VMEM((D, b_per_w), jnp.float32),
            pltpu.SemaphoreType.DMA,
            pltpu.SemaphoreType.DMA,
        ],
        compiler_params=pltpu.CompilerParams(needs_layout_passes=False),
    )
    def gather_kernel(idx_hbm, table_hbm, out_hbm, idx_v, win_a, win_b,
                      out_v, sem_a, sem_b):
        wid = lax.axis_index("s") * info.num_cores + lax.axis_index("c")
        base = wid * b_per_w
        pltpu.sync_copy(idx_hbm.at[pl.ds(base, b_per_w)], idx_v)

        wins = [win_a, win_b]
        sems = [sem_a, sem_b]
        lane_iota = lax.iota(jnp.int32, _LANES)

        def fire(vec, side):
            for k in range(_K):
                c = vec[side * _K + k]
                cb = pl.multiple_of((c >> 7) << 7, 128)
                pltpu.async_copy(
                    table_hbm.at[:, pl.ds(cb, 128)],
                    wins[side].at[:, pl.ds(k * 128, 128)],
                    sems[side],
                )

        def drain(side):
            pltpu.make_async_copy(
                table_hbm.at[:, pl.ds(0, _K * 128)],
                wins[side], sems[side]).wait()

        def extract(vec, b, side):
            win = wins[side]
            for k in range(_K):
                c = vec[side * _K + k]
                lane = jnp.broadcast_to((c & 127) + k * 128, (_LANES,))
                col = jnp.broadcast_to(b * _K + k, (_LANES,))
                for h in range(D // _LANES):
                    rows = lane_iota + h * _LANES
                    vals = plsc.load_gather(win, [rows, lane])
                    plsc.store_scatter(out_v, [rows, col], vals)

        vec0 = idx_v[pl.ds(0, 2 * _K)]
        fire(vec0, 0)
        fire(vec0, 1)

        def body(g, carry):
            vec = idx_v[pl.ds(g * 2 * _K, 2 * _K)]
            nxt = idx_v[pl.ds(
                jnp.minimum(g + 1, n_batches // 2 - 1) * 2 * _K, 2 * _K)]
            for side in range(2):
                b = g * 2 + side
                drain(side)
                extract(vec, b, side)

                @pl.when(b + 2 < n_batches)
                def _():
                    fire(nxt, side)
            return carry

        lax.fori_loop(0, n_batches // 2, body, 0)
        pltpu.sync_copy(out_v, out_hbm.at[:, pl.ds(base, b_per_w)])

    return gather_kernel


def kernel(class_labels, embedding):
    B, = class_labels.shape
    V, D = embedding.shape
    out_t = _build(B, V, D)(class_labels.astype(jnp.int32), embedding.T)
    return out_t.T


# speedup vs baseline: 3.9017x; 1.0087x over previous
import functools

import jax
import jax.numpy as jnp
from jax import lax
from jax.experimental import pallas as pl
from jax.experimental.pallas import tpu as pltpu
from jax.experimental.pallas import tpu_sc as plsc

_K = 8
_LANES = 16


@functools.cache
def _build(B, V, D):
    info = plsc.get_sparse_core_info()
    n_workers = info.num_cores * info.num_subcores
    b_per_w = B // n_workers
    n_batches = b_per_w // _K
    mesh = plsc.VectorSubcoreMesh(core_axis_name="c", subcore_axis_name="s")

    @functools.partial(
        pl.kernel,
        mesh=mesh,
        out_type=jax.ShapeDtypeStruct((D, B), jnp.float32),
        scratch_types=[
            pltpu.VMEM((b_per_w,), jnp.int32),
            pltpu.VMEM((D, _K * 128), jnp.float32),
            pltpu.VMEM((D, _K * 128), jnp.float32),
            pltpu.VMEM((D, b_per_w), jnp.float32),
            pltpu.SemaphoreType.DMA,
            pltpu.SemaphoreType.DMA,
        ],
        compiler_params=pltpu.CompilerParams(needs_layout_passes=False),
    )
    def gather_kernel(idx_hbm, table_hbm, out_hbm, idx_v, win_a, win_b,
                      out_v, sem_a, sem_b):
        wid = lax.axis_index("s") * info.num_cores + lax.axis_index("c")
        base = wid * b_per_w
        pltpu.sync_copy(idx_hbm.at[pl.ds(base, b_per_w)], idx_v)

        wins = [win_a, win_b]
        sems = [sem_a, sem_b]
        lane_iota = lax.iota(jnp.int32, _LANES)

        def fire(vec, side):
            for k in range(_K):
                c = vec[side * _K + k]
                cb = pl.multiple_of((c >> 7) << 7, 128)
                for half in range(2):
                    pltpu.async_copy(
                        table_hbm.at[pl.ds(half * D // 2, D // 2),
                                     pl.ds(cb, 128)],
                        wins[side].at[pl.ds(half * D // 2, D // 2),
                                      pl.ds(k * 128, 128)],
                        sems[side],
                    )

        def drain(side):
            pltpu.make_async_copy(
                table_hbm.at[:, pl.ds(0, _K * 128)],
                wins[side], sems[side]).wait()

        def extract(vec, b, side):
            win = wins[side]
            for k in range(_K):
                c = vec[side * _K + k]
                lane = jnp.broadcast_to((c & 127) + k * 128, (_LANES,))
                col = jnp.broadcast_to(b * _K + k, (_LANES,))
                for h in range(D // _LANES):
                    rows = lane_iota + h * _LANES
                    vals = plsc.load_gather(win, [rows, lane])
                    plsc.store_scatter(out_v, [rows, col], vals)

        vec0 = idx_v[pl.ds(0, 2 * _K)]
        fire(vec0, 0)
        fire(vec0, 1)

        def body(g, carry):
            vec = idx_v[pl.ds(g * 2 * _K, 2 * _K)]
            nxt = idx_v[pl.ds(
                jnp.minimum(g + 1, n_batches // 2 - 1) * 2 * _K, 2 * _K)]
            for side in range(2):
                b = g * 2 + side
                drain(side)
                extract(vec, b, side)

                @pl.when(b + 2 < n_batches)
                def _():
                    fire(nxt, side)
            return carry

        lax.fori_loop(0, n_batches // 2, body, 0)
        pltpu.sync_copy(out_v, out_hbm.at[:, pl.ds(base, b_per_w)])

    return gather_kernel


def kernel(class_labels, embedding):
    B, = class_labels.shape
    V, D = embedding.shape
    out_t = _build(B, V, D)(class_labels.astype(jnp.int32), embedding.T)
    return out_t.T
